# baseline (speedup 1.0000x reference)
import functools

import jax
import jax.numpy as jnp
from jax import lax
from jax.experimental import pallas as pl
from jax.experimental.pallas import tpu as pltpu
from jax.experimental.pallas import tpu_sc as plsc


NW = 32
NPIX = 4 * 512 * 512
PPB = NPIX // 4
PX_PER_W = NPIX // NW
ROWS_PER_W = PX_PER_W // 512
NIT = 512 // 16
GRID_W = 24576
C = 12


def _sc_body(a_hbm, g_hbm, irow_hbm, fxrow_hbm, jrow_hbm, fyrow_hbm, out_hbm,
             a_v, gh_v, g_v, o_v, irow_v, fxrow_v, jrow_v, fyrow_v, sem):
    wid = lax.axis_index("s") * 2 + lax.axis_index("c")
    b = wid // 8
    t = wid % 8
    pltpu.sync_copy(a_hbm.at[pl.ds(b * GRID_W, GRID_W)], a_v)
    pltpu.sync_copy(irow_hbm, irow_v)
    pltpu.sync_copy(fxrow_hbm, fxrow_v)
    pltpu.sync_copy(jrow_hbm, jrow_v)
    pltpu.sync_copy(fyrow_hbm, fyrow_v)

    lane = lax.iota(jnp.int32, 16)

    def row_body(ri, carry):
        row = t * ROWS_PER_W + ri
        p_in_b = row * 512
        pltpu.sync_copy(
            g_hbm.at[pl.ds(pl.multiple_of(b * PPB + p_in_b, 512), 512)], g_v)

        rowv = jnp.full((16,), row, jnp.int32)
        i_off = plsc.load_gather(irow_v, [rowv]) + lane
        fxv = plsc.load_gather(fxrow_v, [rowv])
        wx0 = 1.0 - fxv
        for c in range(C):
            for r in range(8):
                idx = i_off + (c * 2048 + r * 16)
                v0 = plsc.load_gather(a_v, [idx])
                v1 = plsc.load_gather(a_v, [idx + 128])
                gh_v[pl.ds(c * 128 + r * 16, 16)] = wx0 * v0 + fxv * v1

        def it_body(ii, carry2):
            w0 = ii * 16
            g = g_v[pl.ds(pl.multiple_of(w0, 16), 16)]
            j8 = jrow_v[pl.ds(pl.multiple_of(w0, 16), 16)]
            fy = fyrow_v[pl.ds(pl.multiple_of(w0, 16), 16)]
            tz = jnp.clip((g + 1.0) * 3.5, 0.0, 7.0)
            k0 = jnp.minimum(tz.astype(jnp.int32), 6)
            fz = tz - k0.astype(jnp.float32)
            wy0, wz0 = 1.0 - fy, 1.0 - fz
            w00, w01 = wy0 * wz0, wy0 * fz
            w10, w11 = fy * wz0, fy * fz
            base = j8 + k0
            for c in range(C):
                b0 = base + c * 128
                v00 = plsc.load_gather(gh_v, [b0])
                v01 = plsc.load_gather(gh_v, [b0 + 1])
                v10 = plsc.load_gather(gh_v, [b0 + 8])
                v11 = plsc.load_gather(gh_v, [b0 + 9])
                o_v[c, pl.ds(pl.multiple_of(w0, 16), 16)] = (
                    (w00 * v00 + w01 * v01) + (w10 * v10 + w11 * v11))
            return carry2

        lax.fori_loop(0, NIT, it_body, 0)
        copies = [
            pltpu.async_copy(
                o_v.at[c],
                out_hbm.at[pl.ds(
                    pl.multiple_of((b * C + c) * PPB + p_in_b, 512), 512)],
                sem)
            for c in range(C)
        ]
        for cp in copies:
            cp.wait()
        return carry

    lax.fori_loop(0, ROWS_PER_W, row_body, 0)


def _tent_tables(npix, ngrid):
    gx = jnp.linspace(-1.0, 1.0, npix, dtype=jnp.float32)
    tx = jnp.clip((gx + 1.0) * 0.5 * (ngrid - 1), 0.0, float(ngrid - 1))
    i0 = jnp.minimum(jnp.floor(tx), float(ngrid - 2)).astype(jnp.int32)
    f = tx - i0.astype(jnp.float32)
    return i0, f


@jax.jit
def kernel(A, guide):
    bs, H, W, _ = guide.shape
    a_flat = A.reshape(bs * GRID_W)
    g_flat = guide.reshape(bs * H * W)
    i0h, fxh = _tent_tables(H, A.shape[2])
    j0w, fyw = _tent_tables(W, A.shape[3])
    irow = i0h * 128
    jrow = j0w * 8

    mesh = plsc.VectorSubcoreMesh(
        core_axis_name="c", subcore_axis_name="s", num_cores=2, num_subcores=16)
    sc_slice = functools.partial(
        pl.kernel,
        out_type=jax.ShapeDtypeStruct((bs * C * H * W,), jnp.float32),
        mesh=mesh,
        compiler_params=pltpu.CompilerParams(needs_layout_passes=False),
        scratch_types=[
            pltpu.VMEM((GRID_W,), jnp.float32),
            pltpu.VMEM((C * 128,), jnp.float32),
            pltpu.VMEM((512,), jnp.float32),
            pltpu.VMEM((C, 512), jnp.float32),
            pltpu.VMEM((512,), jnp.int32),
            pltpu.VMEM((512,), jnp.float32),
            pltpu.VMEM((512,), jnp.int32),
            pltpu.VMEM((512,), jnp.float32),
            pltpu.SemaphoreType.DMA,
        ],
    )(_sc_body)

    out = sc_slice(a_flat, g_flat, irow, fxh, jrow, fyw)
    return jnp.transpose(out.reshape(bs, C, H, W), (0, 2, 3, 1))


# speedup vs baseline: 2.5625x; 2.5625x over previous
import functools

import jax
import jax.numpy as jnp
from jax import lax
from jax.experimental import pallas as pl
from jax.experimental.pallas import tpu as pltpu
from jax.experimental.pallas import tpu_sc as plsc


C = 12
GRID_W = 24576


B_SC = 1
NW = 32
TPB = NW // B_SC
ROWS_PER_W = 512 * B_SC // NW
NIT = 512 // 16


def _sc_body(a_hbm, g_hbm, irow_hbm, fxrow_hbm, jrow_hbm, fyrow_hbm, out_hbm,
             a_v, gh_v, g_v, o_v, irow_v, fxrow_v, jrow_v, fyrow_v, sem):
    wid = lax.axis_index("s") * 2 + lax.axis_index("c")
    b = wid // TPB
    t = wid % TPB
    pltpu.sync_copy(a_hbm.at[pl.ds(b * GRID_W, GRID_W)], a_v)
    pltpu.sync_copy(irow_hbm, irow_v)
    pltpu.sync_copy(fxrow_hbm, fxrow_v)
    pltpu.sync_copy(jrow_hbm, jrow_v)
    pltpu.sync_copy(fyrow_hbm, fyrow_v)
    lane = lax.iota(jnp.int32, 16)

    def row_body(ri, carry):
        row = t * ROWS_PER_W + ri
        p_in_b = row * 512
        pltpu.sync_copy(
            g_hbm.at[pl.ds(pl.multiple_of(b * 262144 + p_in_b, 512), 512)], g_v)

        rowv = jnp.full((16,), row, jnp.int32)
        i_off = plsc.load_gather(irow_v, [rowv]) + lane
        fxv = plsc.load_gather(fxrow_v, [rowv])
        wx0 = 1.0 - fxv
        for c in range(C):
            for r in range(8):
                idx = i_off + (c * 2048 + r * 16)
                v0 = plsc.load_gather(a_v, [idx])
                v1 = plsc.load_gather(a_v, [idx + 128])
                gh_v[pl.ds(c * 128 + r * 16, 16)] = wx0 * v0 + fxv * v1

        def it_body(ii, carry2):
            w0 = ii * 16
            g = g_v[pl.ds(pl.multiple_of(w0, 16), 16)]
            j8 = jrow_v[pl.ds(pl.multiple_of(w0, 16), 16)]
            fy = fyrow_v[pl.ds(pl.multiple_of(w0, 16), 16)]
            tz = jnp.clip((g + 1.0) * 3.5, 0.0, 7.0)
            k0 = jnp.minimum(tz.astype(jnp.int32), 6)
            fz = tz - k0.astype(jnp.float32)
            wy0, wz0 = 1.0 - fy, 1.0 - fz
            w00, w01 = wy0 * wz0, wy0 * fz
            w10, w11 = fy * wz0, fy * fz
            base = j8 + k0
            for c in range(C):
                b0 = base + c * 128
                v00 = plsc.load_gather(gh_v, [b0])
                v01 = plsc.load_gather(gh_v, [b0 + 1])
                v10 = plsc.load_gather(gh_v, [b0 + 8])
                v11 = plsc.load_gather(gh_v, [b0 + 9])
                o_v[c, pl.ds(pl.multiple_of(w0, 16), 16)] = (
                    (w00 * v00 + w01 * v01) + (w10 * v10 + w11 * v11))
            return carry2

        lax.fori_loop(0, NIT, it_body, 0)
        copies = [
            pltpu.async_copy(o_v.at[c], out_hbm.at[(b * C + c) * 512 + row], sem)
            for c in range(C)
        ]
        for cp in copies:
            cp.wait()
        return carry

    lax.fori_loop(0, ROWS_PER_W, row_body, 0)


def _tent_tables(npix, ngrid):
    gx = jnp.linspace(-1.0, 1.0, npix, dtype=jnp.float32)
    tx = jnp.clip((gx + 1.0) * 0.5 * (ngrid - 1), 0.0, float(ngrid - 1))
    i0 = jnp.minimum(jnp.floor(tx), float(ngrid - 2)).astype(jnp.int32)
    f = tx - i0.astype(jnp.float32)
    return i0, f


def _tent_matrix(npix, ngrid):
    gx = jnp.linspace(-1.0, 1.0, npix, dtype=jnp.float32)
    t = jnp.clip((gx + 1.0) * 0.5 * (ngrid - 1), 0.0, float(ngrid - 1))
    i = jnp.arange(ngrid, dtype=jnp.float32)
    return jnp.maximum(0.0, 1.0 - jnp.abs(t[:, None] - i[None, :]))


BH = 64
CQ = 3
NQ = C // CQ


def _upsample_body(a_ref, v_ref, t_ref):
    t_ref[0] = jnp.dot(a_ref[0], v_ref[...], preferred_element_type=jnp.float32)


def _slice_body(t_ref, u_ref, g_ref, o_ref):
    cmat = jnp.dot(u_ref[...], t_ref[0], preferred_element_type=jnp.float32)
    g = g_ref[0]
    tz = jnp.clip((g + 1.0) * 3.5, 0.0, 7.0)
    wzs = [jnp.maximum(0.0, 1.0 - jnp.abs(tz - z)) for z in range(8)]
    for cc in range(CQ):
        acc = wzs[0] * cmat[:, (cc * 8) * 512:(cc * 8 + 1) * 512]
        for z in range(1, 8):
            acc = acc + wzs[z] * cmat[:, (cc * 8 + z) * 512:(cc * 8 + z + 1) * 512]
        o_ref[0, cc] = acc


def _tc_slice(A, guide2):
    bs, H, W = guide2.shape
    g1, g2, g3 = 16, 16, 8
    U = _tent_matrix(H, g1)
    VT = U.T
    A5 = jnp.transpose(A, (0, 2, 1, 4, 3)).reshape(bs, g1 * C * g3, g2)
    T0 = pl.pallas_call(
        _upsample_body,
        grid=(bs,),
        in_specs=[
            pl.BlockSpec((1, g1 * C * g3, g2), lambda b: (b, 0, 0)),
            pl.BlockSpec((g2, W), lambda b: (0, 0)),
        ],
        out_specs=pl.BlockSpec((1, g1 * C * g3, W), lambda b: (b, 0, 0)),
        out_shape=jax.ShapeDtypeStruct((bs, g1 * C * g3, W), jnp.float32),
    )(A5, VT)
    T = T0.reshape(bs, g1, C * g3 * W)
    O = pl.pallas_call(
        _slice_body,
        grid=(bs, NQ, H // BH),
        in_specs=[
            pl.BlockSpec((1, g1, CQ * g3 * W), lambda b, q, h: (b, 0, q)),
            pl.BlockSpec((BH, g1), lambda b, q, h: (h, 0)),
            pl.BlockSpec((1, BH, W), lambda b, q, h: (b, h, 0)),
        ],
        out_specs=pl.BlockSpec((1, CQ, BH, W), lambda b, q, h: (b, q, h, 0)),
        out_shape=jax.ShapeDtypeStruct((bs, C, H, W), jnp.float32),
    )(T, U, guide2)
    return O


@jax.jit
def kernel(A, guide):
    bs, H, W, _ = guide.shape
    i0h, fxh = _tent_tables(H, A.shape[2])
    j0w, fyw = _tent_tables(W, A.shape[3])

    a_sc = A[:B_SC].reshape(B_SC * GRID_W)
    g_sc = guide[:B_SC].reshape(B_SC * H * W)
    mesh = plsc.VectorSubcoreMesh(
        core_axis_name="c", subcore_axis_name="s", num_cores=2, num_subcores=16)
    sc_slice = functools.partial(
        pl.kernel,
        out_type=jax.ShapeDtypeStruct((B_SC * C * H, W), jnp.float32),
        mesh=mesh,
        compiler_params=pltpu.CompilerParams(needs_layout_passes=False),
        scratch_types=[
            pltpu.VMEM((GRID_W,), jnp.float32),
            pltpu.VMEM((C * 128,), jnp.float32),
            pltpu.VMEM((512,), jnp.float32),
            pltpu.VMEM((C, 512), jnp.float32),
            pltpu.VMEM((512,), jnp.int32),
            pltpu.VMEM((512,), jnp.float32),
            pltpu.VMEM((512,), jnp.int32),
            pltpu.VMEM((512,), jnp.float32),
            pltpu.SemaphoreType.DMA,
        ],
    )(_sc_body)
    out_sc = sc_slice(a_sc, g_sc, i0h * 128, fxh, j0w * 8, fyw)

    out_tc = _tc_slice(A[B_SC:], guide[B_SC:].reshape(bs - B_SC, H, W))

    out = jnp.concatenate(
        [out_sc.reshape(B_SC, C, H, W), out_tc], axis=0)
    return jnp.transpose(out, (0, 2, 3, 1))
